# speedup vs baseline: 22.0423x; 22.0423x over previous
import jax
import jax.numpy as jnp
from jax import lax
from jax.experimental import pallas as pl
from jax.experimental.pallas import tpu as pltpu

_IN_SHAPE = (96, 512)
_LO, _HI = 128, 300
_W = _HI - _LO
_EPS_LOG = 0.001
_COLS = 128
_LOG10_E = 0.4342944819032518


def _body(k, x_ref, o_ref, s_ref):
    x = x_ref[...]
    b = lax.bitcast_convert_type(x, jnp.int32)
    s = jnp.where(b >= 0, b, b ^ jnp.int32(0x7FFFFFFF))
    s_ref[...] = s

    def bis(i, prefix):
        tu = prefix | (jnp.int32(1) << (jnp.int32(31) - i))
        ts = tu ^ jnp.int32(-2147483648)
        cnt = jnp.sum((s_ref[...] < ts).astype(jnp.int32))
        return jnp.where(cnt <= k, tu, prefix)

    prefix = lax.fori_loop(0, 32, bis, jnp.int32(0))
    vs = prefix ^ jnp.int32(-2147483648)
    fb = jnp.where(vs >= 0, vs, vs ^ jnp.int32(0x7FFFFFFF))
    eps = lax.bitcast_convert_type(fb, jnp.float32)

    t = jnp.maximum(eps, jnp.float32(_EPS_LOG))
    z = jnp.log(jnp.maximum(x_ref[...], t)) * jnp.float32(_LOG10_E)
    o_ref[...] = z
    zmin = jnp.min(o_ref[...])
    zmax = jnp.max(o_ref[...])
    o_ref[...] = (o_ref[...] - zmin) / (zmax - zmin)


@jax.jit
def kernel(x):
    xb = x.reshape((-1,) + _IN_SHAPE)
    bsz = xb.shape[0]
    n = bsz * _IN_SHAPE[0] * _W
    rows = n // _COLS
    k = int(0.1 * n)
    xs = xb[:, :, _LO:_HI].reshape(rows, _COLS)

    out = pl.pallas_call(
        lambda xr, orf, sr: _body(k, xr, orf, sr),
        out_shape=jax.ShapeDtypeStruct((rows, _COLS), jnp.float32),
        scratch_shapes=[pltpu.VMEM((rows, _COLS), jnp.int32)],
    )(xs)
    return out.reshape(bsz, _IN_SHAPE[0], _W)
